# baseline (speedup 1.0000x reference)
import functools

import jax
import jax.numpy as jnp
from jax import lax
from jax.experimental import pallas as pl
from jax.experimental.pallas import tpu as pltpu
from jax.experimental.pallas import tpu_sc as plsc

VOCAB = 1000000
EMBED = 64
B_TOTAL = 16384 * 20

_INFO = plsc.get_sparse_core_info()
_NC = _INFO.num_cores
_NS = _INFO.num_subcores
_NW = _NC * _NS
_PER_W = B_TOTAL // _NW
_CHUNK = 512
_NCHUNK = _PER_W // _CHUNK
_CW = _CHUNK * EMBED


def _widen_chunk(src16, dst32):
    iota2 = lax.iota(jnp.int32, 16) * 2

    def body(r, _):
        row = jnp.full((16,), r, dtype=jnp.int32)
        for q in range(EMBED // 32):
            w = plsc.bitcast(src16[r, pl.ds(q * 32, 32)], jnp.int32)
            lo = plsc.bitcast(w << 16, jnp.float32)
            hi = plsc.bitcast(w & jnp.int32(-65536), jnp.float32)
            cols = iota2 + q * 32
            plsc.store_scatter(dst32, [row, cols], lo)
            plsc.store_scatter(dst32, [row, cols + 1], hi)
        return _

    lax.fori_loop(0, _CHUNK, body, None)


def _embed_kernel(idx_hbm, table_hbm, out_hbm, idx_v, r16_0, r16_1, r32_0,
                  r32_1, gsem0, gsem1, osem0, osem1):
    wid = lax.axis_index("s") * _NC + lax.axis_index("c")
    base = wid * _PER_W
    pltpu.sync_copy(idx_hbm.at[pl.ds(base, _PER_W)], idx_v)
    r16 = (r16_0, r16_1)
    r32 = (r32_0, r32_1)
    gsem = (gsem0, gsem1)
    osem = (osem0, osem1)

    def gather(i):
        s = i % 2
        pltpu.async_copy(
            table_hbm.at[idx_v.at[pl.ds(i * _CHUNK, _CHUNK)]], r16[s], gsem[s])

    def gather_wait(i):
        s = i % 2
        pltpu.make_async_copy(
            table_hbm.at[idx_v.at[pl.ds(i * _CHUNK, _CHUNK)]], r16[s],
            gsem[s]).wait()

    def store(i):
        s = i % 2
        return pltpu.async_copy(
            r32[s], out_hbm.at[pl.ds(base + i * _CHUNK, _CHUNK)], osem[s])

    stores = [None, None]
    gather(0)
    for i in range(_NCHUNK):
        s = i % 2
        gather_wait(i)
        if i + 1 < _NCHUNK:
            gather(i + 1)
        if stores[s] is not None:
            stores[s].wait()
        _widen_chunk(r16[s], r32[s])
        stores[s] = store(i)
    stores[0].wait()
    stores[1].wait()


def _sc_gather(idx_flat, table16):
    mesh = plsc.VectorSubcoreMesh(core_axis_name="c", subcore_axis_name="s")
    k = functools.partial(
        pl.kernel,
        mesh=mesh,
        out_type=jax.ShapeDtypeStruct((B_TOTAL, EMBED), jnp.float32),
        scratch_types=[
            pltpu.VMEM((_PER_W,), jnp.int32),
            pltpu.VMEM((_CHUNK, EMBED), jnp.bfloat16),
            pltpu.VMEM((_CHUNK, EMBED), jnp.bfloat16),
            pltpu.VMEM((_CHUNK, EMBED), jnp.float32),
            pltpu.VMEM((_CHUNK, EMBED), jnp.float32),
            pltpu.SemaphoreType.DMA,
            pltpu.SemaphoreType.DMA,
            pltpu.SemaphoreType.DMA,
            pltpu.SemaphoreType.DMA,
        ],
        compiler_params=pltpu.CompilerParams(
            use_tc_tiling_on_sc=False, needs_layout_passes=False),
    )(_embed_kernel)
    return k(idx_flat, table16)


def kernel(input_ids, weight):
    idx_flat = input_ids.reshape(-1).astype(jnp.int32)
    table16 = weight.astype(jnp.bfloat16)
    out = _sc_gather(idx_flat, table16)
    return out.reshape(input_ids.shape + (EMBED,))


# speedup vs baseline: 1.2972x; 1.2972x over previous
import functools

import jax
import jax.numpy as jnp
from jax import lax
from jax.experimental import pallas as pl
from jax.experimental.pallas import tpu as pltpu
from jax.experimental.pallas import tpu_sc as plsc

VOCAB = 1000000
EMBED = 64
B_TOTAL = 16384 * 20

_INFO = plsc.get_sparse_core_info()
_NC = _INFO.num_cores
_NS = _INFO.num_subcores
_NW = _NC * _NS
_PER_W = B_TOTAL // _NW
_CHUNK = 640
_NCHUNK = _PER_W // _CHUNK


def _embed_kernel(idx_hbm, table_hbm, out_hbm, idx_v, rows0, rows1, gsem0,
                  gsem1, osem0, osem1):
    wid = lax.axis_index("s") * _NC + lax.axis_index("c")
    base = wid * _PER_W
    pltpu.sync_copy(idx_hbm.at[pl.ds(base, _PER_W)], idx_v)
    rows = (rows0, rows1)
    gsem = (gsem0, gsem1)
    osem = (osem0, osem1)

    def gather(i):
        s = i % 2
        return pltpu.async_copy(
            table_hbm.at[idx_v.at[pl.ds(i * _CHUNK, _CHUNK)]], rows[s], gsem[s])

    def store(i):
        s = i % 2
        return pltpu.async_copy(
            rows[s], out_hbm.at[pl.ds(base + i * _CHUNK, _CHUNK)], osem[s])

    stores = [None, None]
    gather(0)
    for i in range(_NCHUNK):
        s = i % 2
        pltpu.make_async_copy(
            table_hbm.at[idx_v.at[pl.ds(i * _CHUNK, _CHUNK)]], rows[s],
            gsem[s]).wait()
        if i + 1 < _NCHUNK:
            if stores[(i + 1) % 2] is not None:
                stores[(i + 1) % 2].wait()
            gather(i + 1)
        stores[s] = store(i)
    stores[0].wait()
    stores[1].wait()


def _sc_gather(idx_flat, table):
    mesh = plsc.VectorSubcoreMesh(core_axis_name="c", subcore_axis_name="s")
    k = functools.partial(
        pl.kernel,
        mesh=mesh,
        out_type=jax.ShapeDtypeStruct((B_TOTAL, EMBED), jnp.float32),
        scratch_types=[
            pltpu.VMEM((_PER_W,), jnp.int32),
            pltpu.VMEM((_CHUNK, EMBED), jnp.float32),
            pltpu.VMEM((_CHUNK, EMBED), jnp.float32),
            pltpu.SemaphoreType.DMA,
            pltpu.SemaphoreType.DMA,
            pltpu.SemaphoreType.DMA,
            pltpu.SemaphoreType.DMA,
        ],
        compiler_params=pltpu.CompilerParams(use_tc_tiling_on_sc=False),
    )(_embed_kernel)
    return k(idx_flat, table)


def kernel(input_ids, weight):
    idx_flat = input_ids.reshape(-1).astype(jnp.int32)
    out = _sc_gather(idx_flat, weight)
    return out.reshape(input_ids.shape + (EMBED,))
